# Initial kernel scaffold; baseline (speedup 1.0000x reference)
import jax
import jax.numpy as jnp
from jax.experimental import pallas as pl


def kernel(pred, target):
    raise NotImplementedError("write your pallas kernel here")


# speedup vs baseline: 3.1686x; 3.1686x over previous
import jax
import jax.numpy as jnp
from jax.experimental import pallas as pl
from jax.experimental.pallas import tpu as pltpu

_SMOOTHING = 0.1
_N_CLASSES = 1000
_A = _SMOOTHING / (_N_CLASSES - 1)
_B = (1.0 - _SMOOTHING) - _A

_ROWS_PER_BLOCK = 1024


def _body(pred_ref, target_ref, out_ref):
    i = pl.program_id(0)
    pred = pred_ref[...]
    t = target_ref[...]
    m = jnp.max(pred, axis=-1, keepdims=True)
    s = jnp.sum(jnp.exp(pred - m), axis=-1)
    lse = m[:, 0] + jnp.log(s)
    cols = jax.lax.broadcasted_iota(jnp.int32, pred.shape, 1)
    w = _A + _B * (cols == t[:, None]).astype(jnp.float32)
    ws = jnp.sum(w * pred, axis=-1)
    part = jnp.sum(lse - ws)

    @pl.when(i == 0)
    def _init():
        out_ref[0, 0] = 0.0

    out_ref[0, 0] += part


def kernel(pred, target):
    n_rows = pred.shape[0]
    grid = n_rows // _ROWS_PER_BLOCK
    total = pl.pallas_call(
        _body,
        grid=(grid,),
        in_specs=[
            pl.BlockSpec((_ROWS_PER_BLOCK, _N_CLASSES), lambda i: (i, 0)),
            pl.BlockSpec((_ROWS_PER_BLOCK,), lambda i: (i,)),
        ],
        out_specs=pl.BlockSpec(
            (1, 1), lambda i: (0, 0), memory_space=pltpu.SMEM
        ),
        out_shape=jax.ShapeDtypeStruct((1, 1), jnp.float32),
    )(pred, target.astype(jnp.int32))
    return total[0, 0] / n_rows
